# baseline (speedup 1.0000x reference)
import functools

import jax
import jax.numpy as jnp
from jax.experimental import pallas as pl
from jax.experimental.pallas import tpu as pltpu

_BN_EPS = 1e-5
_VMEM_LIMIT = 64 * 1024 * 1024
_MM_DTYPE = jnp.bfloat16


def _emit_patches(xm, xq_ref, p_ref, *, C, W, HW):
    zrow = jnp.zeros((C, 2 * W), dtype=xq_ref.dtype)
    xq_ref[:, : 2 * W] = zrow
    xq_ref[:, 2 * W: 2 * W + HW] = xm
    xq_ref[:, 2 * W + HW:] = zrow
    lane = jax.lax.broadcasted_iota(jnp.int32, (1, HW), 1) % W
    zero = jnp.zeros((), dtype=xq_ref.dtype)
    for kh in range(3):
        for kw in range(3):
            t = kh * 3 + kw
            start = (kh + 1) * W + kw - 1
            slab = xq_ref[:, start:start + HW]
            if kw == 0:
                slab = jnp.where(lane != 0, slab, zero)
            elif kw == 2:
                slab = jnp.where(lane != W - 1, slab, zero)
            p_ref[t * C:(t + 1) * C, :] = slab


def _stats_kernel(xm_ref, w_ref, s_ref, q_ref, xq_ref, p_ref, *, C, W, HW):
    _emit_patches(xm_ref[0], xq_ref, p_ref, C=C, W=W, HW=HW)
    y = jnp.dot(w_ref[...], p_ref[...],
                preferred_element_type=jnp.float32)
    s_ref[...] = jnp.sum(y, axis=1, keepdims=True)[None]
    q_ref[...] = jnp.sum(y * y, axis=1, keepdims=True)[None]


def _conv_bn_kernel(xm_ref, w_ref, sc_ref, sh_ref, o_ref, xq_ref, p_ref,
                    *, C, W, HW):
    _emit_patches(xm_ref[0], xq_ref, p_ref, C=C, W=W, HW=HW)
    y = jnp.dot(w_ref[...], p_ref[...],
                preferred_element_type=jnp.float32)
    o = y * sc_ref[...] + sh_ref[...]
    o_ref[...] = jnp.maximum(o, 0.0)[None]


def _mask_tile_kernel(m_ref, nm_ref, *, rep):
    nm_ref[...] = jnp.concatenate([m_ref[0]] * rep, axis=0)[None]


def kernel(x, mask, weight, bias):
    del bias
    N, C, H, W = x.shape
    Cout = weight.shape[0]
    HW = H * W
    M = N * HW
    KK = 9 * C
    JQ = HW + 4 * W
    rep = Cout // C

    xm4 = (x * mask).astype(_MM_DTYPE).reshape(N, C, HW)
    wmat = weight.transpose(0, 2, 3, 1).reshape(Cout, KK).astype(_MM_DTYPE)

    scratches = [
        pltpu.VMEM((C, JQ), _MM_DTYPE),
        pltpu.VMEM((KK, HW), _MM_DTYPE),
    ]
    params = pltpu.CompilerParams(
        dimension_semantics=("parallel",),
        vmem_limit_bytes=_VMEM_LIMIT,
    )

    stats_body = functools.partial(_stats_kernel, C=C, W=W, HW=HW)
    s, q = pl.pallas_call(
        stats_body,
        out_shape=(
            jax.ShapeDtypeStruct((N, Cout, 1), jnp.float32),
            jax.ShapeDtypeStruct((N, Cout, 1), jnp.float32),
        ),
        grid=(N,),
        in_specs=[
            pl.BlockSpec((1, C, HW), lambda i: (i, 0, 0)),
            pl.BlockSpec((Cout, KK), lambda i: (0, 0)),
        ],
        out_specs=(
            pl.BlockSpec((1, Cout, 1), lambda i: (i, 0, 0)),
            pl.BlockSpec((1, Cout, 1), lambda i: (i, 0, 0)),
        ),
        scratch_shapes=scratches,
        compiler_params=params,
    )(xm4, wmat)

    mean = jnp.sum(s, axis=0) / M
    var = jnp.maximum(jnp.sum(q, axis=0) / M - mean * mean, 0.0)
    rstd = jax.lax.rsqrt(var + _BN_EPS)
    scale = rstd
    shift = -mean * rstd

    bn_body = functools.partial(_conv_bn_kernel, C=C, W=W, HW=HW)
    o = pl.pallas_call(
        bn_body,
        out_shape=jax.ShapeDtypeStruct((N, Cout, HW), jnp.float32),
        grid=(N,),
        in_specs=[
            pl.BlockSpec((1, C, HW), lambda i: (i, 0, 0)),
            pl.BlockSpec((Cout, KK), lambda i: (0, 0)),
            pl.BlockSpec((Cout, 1), lambda i: (0, 0)),
            pl.BlockSpec((Cout, 1), lambda i: (0, 0)),
        ],
        out_specs=pl.BlockSpec((1, Cout, HW), lambda i: (i, 0, 0)),
        scratch_shapes=scratches,
        compiler_params=params,
    )(xm4, wmat, scale, shift)

    nm = pl.pallas_call(
        functools.partial(_mask_tile_kernel, rep=rep),
        out_shape=jax.ShapeDtypeStruct((N, Cout, H, W), jnp.float32),
        grid=(N,),
        in_specs=[pl.BlockSpec((1, C, H, W), lambda i: (i, 0, 0, 0))],
        out_specs=pl.BlockSpec((1, Cout, H, W), lambda i: (i, 0, 0, 0)),
        compiler_params=params,
    )(mask)

    return o.reshape(N, Cout, H, W), nm


# speedup vs baseline: 1.2809x; 1.2809x over previous
import functools

import jax
import jax.numpy as jnp
from jax.experimental import pallas as pl
from jax.experimental.pallas import tpu as pltpu

_BN_EPS = 1e-5
_VMEM_LIMIT = 64 * 1024 * 1024
_MM_DTYPE = jnp.bfloat16


def _emit_patches(xq, p_ref, *, C, W, HW):
    lane = jax.lax.broadcasted_iota(jnp.int32, (1, HW), 1) % W
    zero = jnp.zeros((), dtype=xq.dtype)
    for kh in range(3):
        for kw in range(3):
            t = kh * 3 + kw
            start = (kh + 1) * W + kw - 1
            slab = xq[:, start:start + HW]
            if kw == 0:
                slab = jnp.where(lane != 0, slab, zero)
            elif kw == 2:
                slab = jnp.where(lane != W - 1, slab, zero)
            p_ref[t * C:(t + 1) * C, :] = slab


def _stats_mask_kernel(x_ref, m_ref, w_ref, xq_ref, s_ref, q_ref, nm_ref,
                       p_ref, *, C, W, HW, rep):
    xm = (x_ref[0] * m_ref[0]).astype(xq_ref.dtype)
    zrow = jnp.zeros((C, 2 * W), dtype=xq_ref.dtype)
    xq_ref[0, :, : 2 * W] = zrow
    xq_ref[0, :, 2 * W: 2 * W + HW] = xm
    xq_ref[0, :, 2 * W + HW:] = zrow
    _emit_patches(xq_ref[0], p_ref, C=C, W=W, HW=HW)
    y = jnp.dot(w_ref[...], p_ref[...],
                preferred_element_type=jnp.float32)
    s_ref[...] = jnp.sum(y, axis=1, keepdims=True)[None]
    q_ref[...] = jnp.sum(y * y, axis=1, keepdims=True)[None]
    m = m_ref[0]
    nm_ref[...] = jnp.concatenate([m] * rep, axis=0)[None]


def _conv_bn_kernel(xq_ref, w_ref, sc_ref, sh_ref, o_ref, p_ref, *, C, W, HW):
    _emit_patches(xq_ref[0], p_ref, C=C, W=W, HW=HW)
    y = jnp.dot(w_ref[...], p_ref[...],
                preferred_element_type=jnp.float32)
    o = y * sc_ref[...] + sh_ref[...]
    o_ref[...] = jnp.maximum(o, 0.0)[None]


def kernel(x, mask, weight, bias):
    del bias
    N, C, H, W = x.shape
    Cout = weight.shape[0]
    HW = H * W
    M = N * HW
    KK = 9 * C
    JQ = HW + 4 * W
    rep = Cout // C

    x4 = x.reshape(N, C, HW)
    m4 = mask.reshape(N, C, HW)
    wmat = weight.transpose(0, 2, 3, 1).reshape(Cout, KK).astype(_MM_DTYPE)

    patches_scratch = [pltpu.VMEM((KK, HW), _MM_DTYPE)]
    params = pltpu.CompilerParams(
        dimension_semantics=("parallel",),
        vmem_limit_bytes=_VMEM_LIMIT,
    )

    stats_body = functools.partial(_stats_mask_kernel, C=C, W=W, HW=HW, rep=rep)
    xq, s, q, nm = pl.pallas_call(
        stats_body,
        out_shape=(
            jax.ShapeDtypeStruct((N, C, JQ), _MM_DTYPE),
            jax.ShapeDtypeStruct((N, Cout, 1), jnp.float32),
            jax.ShapeDtypeStruct((N, Cout, 1), jnp.float32),
            jax.ShapeDtypeStruct((N, Cout, HW), jnp.float32),
        ),
        grid=(N,),
        in_specs=[
            pl.BlockSpec((1, C, HW), lambda i: (i, 0, 0)),
            pl.BlockSpec((1, C, HW), lambda i: (i, 0, 0)),
            pl.BlockSpec((Cout, KK), lambda i: (0, 0)),
        ],
        out_specs=(
            pl.BlockSpec((1, C, JQ), lambda i: (i, 0, 0)),
            pl.BlockSpec((1, Cout, 1), lambda i: (i, 0, 0)),
            pl.BlockSpec((1, Cout, 1), lambda i: (i, 0, 0)),
            pl.BlockSpec((1, Cout, HW), lambda i: (i, 0, 0)),
        ),
        scratch_shapes=patches_scratch,
        compiler_params=params,
    )(x4, m4, wmat)

    mean = jnp.sum(s, axis=0) / M
    var = jnp.maximum(jnp.sum(q, axis=0) / M - mean * mean, 0.0)
    rstd = jax.lax.rsqrt(var + _BN_EPS)
    scale = rstd
    shift = -mean * rstd

    bn_body = functools.partial(_conv_bn_kernel, C=C, W=W, HW=HW)
    o = pl.pallas_call(
        bn_body,
        out_shape=jax.ShapeDtypeStruct((N, Cout, HW), jnp.float32),
        grid=(N,),
        in_specs=[
            pl.BlockSpec((1, C, JQ), lambda i: (i, 0, 0)),
            pl.BlockSpec((Cout, KK), lambda i: (0, 0)),
            pl.BlockSpec((Cout, 1), lambda i: (0, 0)),
            pl.BlockSpec((Cout, 1), lambda i: (0, 0)),
        ],
        out_specs=pl.BlockSpec((1, Cout, HW), lambda i: (i, 0, 0)),
        scratch_shapes=patches_scratch,
        compiler_params=params,
    )(xq, wmat, scale, shift)

    return o.reshape(N, Cout, H, W), nm.reshape(N, Cout, H, W)


# speedup vs baseline: 1.3660x; 1.0664x over previous
import functools

import jax
import jax.numpy as jnp
from jax.experimental import pallas as pl
from jax.experimental.pallas import tpu as pltpu

_BN_EPS = 1e-5
_VMEM_LIMIT = 64 * 1024 * 1024
_MM_DTYPE = jnp.bfloat16


def _emit_patches(xq, p_ref, *, C, W, HW):
    lane = jax.lax.broadcasted_iota(jnp.int32, (1, HW), 1) % W
    zero = jnp.zeros((), dtype=xq.dtype)
    for kh in range(3):
        for kw in range(3):
            t = kh * 3 + kw
            start = (kh + 1) * W + kw - 1
            slab = xq[:, start:start + HW]
            if kw == 0:
                slab = jnp.where(lane != 0, slab, zero)
            elif kw == 2:
                slab = jnp.where(lane != W - 1, slab, zero)
            p_ref[t * C:(t + 1) * C, :] = slab


def _stats_mask_kernel(x_ref, m_ref, w_ref, xq_ref, s_ref, q_ref, nm_ref,
                       p_ref, *, C, W, HW, rep):
    xm = (x_ref[0] * m_ref[0]).astype(xq_ref.dtype)
    zrow = jnp.zeros((C, 2 * W), dtype=xq_ref.dtype)
    xq_ref[0, :, : 2 * W] = zrow
    xq_ref[0, :, 2 * W: 2 * W + HW] = xm
    xq_ref[0, :, 2 * W + HW:] = zrow
    _emit_patches(xq_ref[0], p_ref, C=C, W=W, HW=HW)
    y = jnp.dot(w_ref[...], p_ref[...],
                preferred_element_type=jnp.float32)
    s_ref[...] = jnp.sum(y, axis=1, keepdims=True)[None]
    q_ref[...] = jnp.sum(y * y, axis=1, keepdims=True)[None]
    m = m_ref[0]
    nm_ref[...] = jnp.concatenate([m] * rep, axis=0).astype(nm_ref.dtype)[None]


def _conv_bn_kernel(xq_ref, w_ref, sc_ref, sh_ref, o_ref, p_ref, *, C, W, HW):
    _emit_patches(xq_ref[0], p_ref, C=C, W=W, HW=HW)
    y = jnp.dot(w_ref[...], p_ref[...],
                preferred_element_type=jnp.float32)
    o = y * sc_ref[...] + sh_ref[...]
    o_ref[...] = jnp.maximum(o, 0.0).astype(o_ref.dtype)[None]


def kernel(x, mask, weight, bias):
    del bias
    N, C, H, W = x.shape
    Cout = weight.shape[0]
    HW = H * W
    M = N * HW
    KK = 9 * C
    JQ = HW + 4 * W
    rep = Cout // C

    x4 = x.reshape(N, C, HW)
    m4 = mask.reshape(N, C, HW)
    wmat = weight.transpose(0, 2, 3, 1).reshape(Cout, KK).astype(_MM_DTYPE)

    patches_scratch = [pltpu.VMEM((KK, HW), _MM_DTYPE)]
    params = pltpu.CompilerParams(
        dimension_semantics=("parallel",),
        vmem_limit_bytes=_VMEM_LIMIT,
    )

    stats_body = functools.partial(_stats_mask_kernel, C=C, W=W, HW=HW, rep=rep)
    xq, s, q, nm = pl.pallas_call(
        stats_body,
        out_shape=(
            jax.ShapeDtypeStruct((N, C, JQ), _MM_DTYPE),
            jax.ShapeDtypeStruct((N, Cout, 1), jnp.float32),
            jax.ShapeDtypeStruct((N, Cout, 1), jnp.float32),
            jax.ShapeDtypeStruct((N, Cout, HW), jnp.bfloat16),
        ),
        grid=(N,),
        in_specs=[
            pl.BlockSpec((1, C, HW), lambda i: (i, 0, 0)),
            pl.BlockSpec((1, C, HW), lambda i: (i, 0, 0)),
            pl.BlockSpec((Cout, KK), lambda i: (0, 0)),
        ],
        out_specs=(
            pl.BlockSpec((1, C, JQ), lambda i: (i, 0, 0)),
            pl.BlockSpec((1, Cout, 1), lambda i: (i, 0, 0)),
            pl.BlockSpec((1, Cout, 1), lambda i: (i, 0, 0)),
            pl.BlockSpec((1, Cout, HW), lambda i: (i, 0, 0)),
        ),
        scratch_shapes=patches_scratch,
        compiler_params=params,
    )(x4, m4, wmat)

    mean = jnp.sum(s, axis=0) / M
    var = jnp.maximum(jnp.sum(q, axis=0) / M - mean * mean, 0.0)
    rstd = jax.lax.rsqrt(var + _BN_EPS)
    scale = rstd
    shift = -mean * rstd

    bn_body = functools.partial(_conv_bn_kernel, C=C, W=W, HW=HW)
    o = pl.pallas_call(
        bn_body,
        out_shape=jax.ShapeDtypeStruct((N, Cout, HW), jnp.bfloat16),
        grid=(N,),
        in_specs=[
            pl.BlockSpec((1, C, JQ), lambda i: (i, 0, 0)),
            pl.BlockSpec((Cout, KK), lambda i: (0, 0)),
            pl.BlockSpec((Cout, 1), lambda i: (0, 0)),
            pl.BlockSpec((Cout, 1), lambda i: (0, 0)),
        ],
        out_specs=pl.BlockSpec((1, Cout, HW), lambda i: (i, 0, 0)),
        scratch_shapes=patches_scratch,
        compiler_params=params,
    )(xq, wmat, scale, shift)

    h = o.reshape(N, Cout, H, W).astype(jnp.float32)
    new_mask = nm.reshape(N, Cout, H, W).astype(jnp.float32)
    return h, new_mask


# speedup vs baseline: 1.4519x; 1.0629x over previous
import functools

import jax
import jax.numpy as jnp
from jax.experimental import pallas as pl
from jax.experimental.pallas import tpu as pltpu

_BN_EPS = 1e-5
_VMEM_LIMIT = 64 * 1024 * 1024
_MM_DTYPE = jnp.float32


def _emit_patches(xq, p_ref, *, C, W, HW):
    lane = jax.lax.broadcasted_iota(jnp.int32, (1, HW), 1) % W
    zero = jnp.zeros((), dtype=xq.dtype)
    for kh in range(3):
        for kw in range(3):
            t = kh * 3 + kw
            start = (kh + 1) * W + kw - 1
            slab = xq[:, start:start + HW]
            if kw == 0:
                slab = jnp.where(lane != 0, slab, zero)
            elif kw == 2:
                slab = jnp.where(lane != W - 1, slab, zero)
            p_ref[t * C:(t + 1) * C, :] = slab


def _stats_mask_kernel(x_ref, m_ref, w_ref, xq_ref, s_ref, q_ref, nm_ref,
                       p_ref, *, C, W, HW, rep):
    xm = (x_ref[0] * m_ref[0]).astype(xq_ref.dtype)
    zrow = jnp.zeros((C, 2 * W), dtype=xq_ref.dtype)
    xq_ref[0, :, : 2 * W] = zrow
    xq_ref[0, :, 2 * W: 2 * W + HW] = xm
    xq_ref[0, :, 2 * W + HW:] = zrow
    _emit_patches(xq_ref[0], p_ref, C=C, W=W, HW=HW)
    y = jnp.dot(w_ref[...], p_ref[...],
                preferred_element_type=jnp.float32)
    s_ref[...] = jnp.sum(y, axis=1, keepdims=True)[None]
    q_ref[...] = jnp.sum(y * y, axis=1, keepdims=True)[None]
    m = m_ref[0]
    nm_ref[...] = jnp.concatenate([m] * rep, axis=0).astype(nm_ref.dtype)[None]


def _conv_bn_kernel(xq_ref, w_ref, sc_ref, sh_ref, o_ref, p_ref, *, C, W, HW):
    _emit_patches(xq_ref[0], p_ref, C=C, W=W, HW=HW)
    y = jnp.dot(w_ref[...], p_ref[...],
                preferred_element_type=jnp.float32)
    o = y * sc_ref[...] + sh_ref[...]
    o_ref[...] = jnp.maximum(o, 0.0).astype(o_ref.dtype)[None]


def kernel(x, mask, weight, bias):
    del bias
    N, C, H, W = x.shape
    Cout = weight.shape[0]
    HW = H * W
    M = N * HW
    KK = 9 * C
    JQ = HW + 4 * W
    rep = Cout // C

    x4 = x.reshape(N, C, HW)
    m4 = mask.reshape(N, C, HW)
    wmat = weight.transpose(0, 2, 3, 1).reshape(Cout, KK).astype(_MM_DTYPE)

    patches_scratch = [pltpu.VMEM((KK, HW), _MM_DTYPE)]
    params = pltpu.CompilerParams(
        dimension_semantics=("parallel",),
        vmem_limit_bytes=_VMEM_LIMIT,
    )

    stats_body = functools.partial(_stats_mask_kernel, C=C, W=W, HW=HW, rep=rep)
    xq, s, q, nm = pl.pallas_call(
        stats_body,
        out_shape=(
            jax.ShapeDtypeStruct((N, C, JQ), _MM_DTYPE),
            jax.ShapeDtypeStruct((N, Cout, 1), jnp.float32),
            jax.ShapeDtypeStruct((N, Cout, 1), jnp.float32),
            jax.ShapeDtypeStruct((N, Cout, HW), jnp.bfloat16),
        ),
        grid=(N,),
        in_specs=[
            pl.BlockSpec((1, C, HW), lambda i: (i, 0, 0)),
            pl.BlockSpec((1, C, HW), lambda i: (i, 0, 0)),
            pl.BlockSpec((Cout, KK), lambda i: (0, 0)),
        ],
        out_specs=(
            pl.BlockSpec((1, C, JQ), lambda i: (i, 0, 0)),
            pl.BlockSpec((1, Cout, 1), lambda i: (i, 0, 0)),
            pl.BlockSpec((1, Cout, 1), lambda i: (i, 0, 0)),
            pl.BlockSpec((1, Cout, HW), lambda i: (i, 0, 0)),
        ),
        scratch_shapes=patches_scratch,
        compiler_params=params,
    )(x4, m4, wmat)

    mean = jnp.sum(s, axis=0) / M
    var = jnp.maximum(jnp.sum(q, axis=0) / M - mean * mean, 0.0)
    rstd = jax.lax.rsqrt(var + _BN_EPS)
    scale = rstd
    shift = -mean * rstd

    bn_body = functools.partial(_conv_bn_kernel, C=C, W=W, HW=HW)
    o = pl.pallas_call(
        bn_body,
        out_shape=jax.ShapeDtypeStruct((N, Cout, HW), jnp.bfloat16),
        grid=(N,),
        in_specs=[
            pl.BlockSpec((1, C, JQ), lambda i: (i, 0, 0)),
            pl.BlockSpec((Cout, KK), lambda i: (0, 0)),
            pl.BlockSpec((Cout, 1), lambda i: (0, 0)),
            pl.BlockSpec((Cout, 1), lambda i: (0, 0)),
        ],
        out_specs=pl.BlockSpec((1, Cout, HW), lambda i: (i, 0, 0)),
        scratch_shapes=patches_scratch,
        compiler_params=params,
    )(xq, wmat, scale, shift)

    h = o.reshape(N, Cout, H, W).astype(jnp.float32)
    new_mask = nm.reshape(N, Cout, H, W).astype(jnp.float32)
    return h, new_mask
